# baseline (speedup 1.0000x reference)
import functools

import jax
import jax.numpy as jnp
import numpy as np
from jax import lax
from jax.experimental import pallas as pl
from jax.experimental.pallas import tpu as pltpu
from jax.experimental.pallas import tpu_sc as plsc

_H = 125
_HC = 25
_LANES = 16
_SIZES = (61, 31, 15)
_SCALES = (4, 2, 1)
_PADS = (16, 8, 4)
_GROUPS = ((0, 0, 24), (1, 24, 4), (2, 28, 2))
_WID_SMALL_BANDS = 30
_WID_P0_BANDS = 31


def _clips_low(k):
    s, p = _SCALES[k], _PADS[k]
    return [p - s * v for v in range(4) if p - s * v > 0]


def _clips_high(k):
    n, s, p = _SIZES[k], _SCALES[k], _PADS[k]
    return sorted({s * v - (_H + p - n) for v in range(25)
                   if s * v - (_H + p - n) > 0})


def _to_rows(x):
    B, C, H, W = x.shape
    y = x.reshape(B, 2, 128, H, W).transpose(3, 4, 1, 0, 2)
    return y.reshape(H * W, 8, 128)


def _from_rows(y, H, W):
    z = y.reshape(H, W, 2, 4, 128).transpose(3, 2, 4, 0, 1)
    return z.reshape(4, 256, H, W)


def _body(ffr, cor, anch, zsrc, q0, q1, q2, q3, av, *, nsub, nworkers):
    outs = (q0, q1, q2)
    wid = lax.axis_index("c") * nsub + lax.axis_index("s")

    pltpu.sync_copy(anch, av)
    lane = lax.iota(jnp.int32, _LANES)
    r = jnp.max(plsc.load_gather(av, [lane]))
    c = jnp.max(plsc.load_gather(av, [_LANES + lane]))

    geo = []
    for k in range(3):
        n, scale, pad = _SIZES[k], _SCALES[k], _PADS[k]
        r0 = scale * r - pad
        c0 = scale * c - pad
        rlo = jnp.clip(-r0, 0, n)
        rhi = jnp.clip(r0 + n - _H, 0, n)
        clo = jnp.clip(-c0, 0, n)
        chi = jnp.clip(c0 + n - _H, 0, n)
        nrows = n - rlo - rhi
        ncols = n - clo - chi
        sr0 = jnp.maximum(r0, 0)
        sc0 = jnp.maximum(c0, 0)
        geo.append((rlo, rhi, clo, chi, nrows, ncols, sr0, sc0))

    for (k, w0, nw) in _GROUPS:
        n = _SIZES[k]
        (rlo, rhi, clo, chi, nrows, ncols, sr0, sc0) = geo[k]
        qk = outs[k]
        cs = -(-n // nw)
        t = wid - w0
        on = jnp.logical_and(wid >= w0, wid < w0 + nw)
        for ncv in sorted({n} | {n - v for v in _clips_low(k)}
                          | {n - v for v in _clips_high(k)}):

            @pl.when(jnp.logical_and(on, ncols == ncv))
            def _(ncv=ncv, n=n, cs=cs, t=t, qk=qk, rlo=rlo, clo=clo,
                  nrows=nrows, sr0=sr0, sc0=sc0):
                for j in range(cs):
                    rr = jnp.clip(t * cs + j, 0, nrows - 1)
                    pltpu.sync_copy(
                        ffr.at[pl.ds((sr0 + rr) * _H + sc0, ncv)],
                        qk.at[pl.ds((rlo + rr) * n + clo, ncv)])

    def bands(k, on):
        n = _SIZES[k]
        (rlo, rhi, clo, chi, nrows, ncols, sr0, sc0) = geo[k]
        qk = outs[k]
        for v in _clips_low(k):

            @pl.when(jnp.logical_and(on, rlo == v))
            def _(v=v, n=n, qk=qk):
                pltpu.sync_copy(zsrc.at[pl.ds(0, v * n)],
                                qk.at[pl.ds(0, v * n)])

            @pl.when(jnp.logical_and(on, clo == v))
            def _(v=v, n=n, qk=qk):
                def row_fill(row, carry):
                    pltpu.sync_copy(zsrc.at[pl.ds(0, v)],
                                    qk.at[pl.ds(row * n, v)])
                    return carry
                lax.fori_loop(0, n, row_fill, jnp.int32(0))

        for v in _clips_high(k):

            @pl.when(jnp.logical_and(on, rhi == v))
            def _(v=v, n=n, qk=qk):
                pltpu.sync_copy(zsrc.at[pl.ds(0, v * n)],
                                qk.at[pl.ds((n - v) * n, v * n)])

            @pl.when(jnp.logical_and(on, chi == v))
            def _(v=v, n=n, qk=qk):
                def row_fill(row, carry):
                    pltpu.sync_copy(zsrc.at[pl.ds(0, v)],
                                    qk.at[pl.ds(row * n + n - v, v)])
                    return carry
                lax.fori_loop(0, n, row_fill, jnp.int32(0))

    bands(0, wid == _WID_P0_BANDS)
    bands(1, wid == _WID_SMALL_BANDS)
    bands(2, wid == _WID_SMALL_BANDS)

    @pl.when(wid == _WID_SMALL_BANDS)
    def _():
        pltpu.sync_copy(cor.at[pl.ds(_HC * r + c, 1)], q3)


def kernel(full_feature, corr_feature, anchor):
    ffr = _to_rows(full_feature)
    cor = _to_rows(corr_feature)
    a32 = anchor.astype(jnp.int32)
    anch = jnp.concatenate([
        jnp.broadcast_to(a32[0], (_LANES,)),
        jnp.broadcast_to(a32[1], (_LANES,)),
    ])
    f32 = jnp.float32
    zsrc = jnp.asarray(np.zeros((16 * 61, 8, 128), np.float32))

    mesh = plsc.VectorSubcoreMesh(core_axis_name="c", subcore_axis_name="s")
    nworkers = mesh.num_cores * mesh.num_subcores

    run = pl.kernel(
        functools.partial(_body, nsub=mesh.num_subcores, nworkers=nworkers),
        out_type=(
            jax.ShapeDtypeStruct((61 * 61, 8, 128), f32),
            jax.ShapeDtypeStruct((31 * 31, 8, 128), f32),
            jax.ShapeDtypeStruct((15 * 15, 8, 128), f32),
            jax.ShapeDtypeStruct((1, 8, 128), f32),
        ),
        mesh=mesh,
        compiler_params=pltpu.CompilerParams(use_tc_tiling_on_sc=True,
                                             needs_layout_passes=False),
        scratch_types=[
            pltpu.VMEM((2 * _LANES,), jnp.int32),
        ],
    )
    q0, q1, q2, q3 = run(ffr, cor, anch, zsrc)

    return (_from_rows(q0, 61, 61), _from_rows(q1, 31, 31),
            _from_rows(q2, 15, 15), _from_rows(q3, 1, 1))


# speedup vs baseline: 1.4908x; 1.4908x over previous
import functools

import jax
import jax.numpy as jnp
import numpy as np
from jax import lax
from jax.experimental import pallas as pl
from jax.experimental.pallas import tpu as pltpu
from jax.experimental.pallas import tpu_sc as plsc

_H = 125
_HC = 25
_LANES = 16
_U = 31
_SIZES = (61, 31, 15)
_SCALES = (4, 2, 1)
_PADS = (16, 8, 4)
_TYPES = ((1, 0, 31), (2, 0, 15))
_STARTS = (0, 31)
_NUNITS = 46


def _to_rows(x):
    B, C, H, W = x.shape
    y = x.reshape(B, 2, 128, H, W).transpose(3, 4, 1, 0, 2)
    return y.reshape(H * W, 8, 128)


def _from_rows(y, H, W):
    z = y.reshape(H, W, 2, 4, 128).transpose(3, 2, 4, 0, 1)
    return z.reshape(4, 256, H, W)


def _body(ffr, cor, anch, zin, q1, q2, q3,
          av, bufa, bufb, zer, idxa, idxb,
          sem_ga, sem_gb, sem_o, sem_m, *, nsub, nworkers):
    outs = {1: q1, 2: q2}
    bufs = (bufa, bufb)
    idxs = (idxa, idxb)
    gsems = (sem_ga, sem_gb)

    wid = lax.axis_index("c") * nsub + lax.axis_index("s")

    pltpu.sync_copy(zin, zer)

    pltpu.sync_copy(anch, av)
    lane = lax.iota(jnp.int32, _LANES)
    r = jnp.max(plsc.load_gather(av, [lane]))
    c = jnp.max(plsc.load_gather(av, [_LANES + lane]))

    zero16 = jnp.broadcast_to(jnp.int32(0), (_LANES,))
    for ib in idxs:
        plsc.store_scatter(ib, [lane], zero16)
        plsc.store_scatter(ib, [_LANES + lane], zero16,
                           mask=(_LANES + lane) < _U)

    tconst = []
    for (k, col0, npx) in _TYPES:
        size, scale, pad = _SIZES[k], _SCALES[k], _PADS[k]
        r0 = scale * r - pad
        c0 = scale * c - pad
        chunks = []
        for j in range((npx + _LANES - 1) // _LANES):
            oc = _LANES * j + lane
            chunks.append((oc, jnp.clip(c0 + col0 + oc, 0, _H - 1),
                           oc < npx))
        nlo = jnp.clip(-c0 - col0, 0, npx)
        nhi = jnp.clip(c0 + col0 + npx - _H, 0, npx)
        tconst.append((k, col0, npx, r0, chunks, nlo, nhi))

    def unit_u(rid, t):
        return rid - _STARTS[t]

    def build_idx(slot_rid, par):
        for t, (k, col0, npx, r0, chunks, nlo, nhi) in enumerate(tconst):
            lo = _STARTS[t]
            hi = _STARTS[t + 1] if t + 1 < len(_STARTS) else _NUNITS
            u = unit_u(slot_rid, t)
            srow = r0 + u
            valid = jnp.logical_and(srow >= 0, srow < _H)
            on = jnp.logical_and(
                jnp.logical_and(slot_rid >= lo, slot_rid < hi), valid)

            @pl.when(on)
            def _():
                rbase = srow * _H
                for (oc, ccl, m) in chunks:
                    plsc.store_scatter(idxs[par], [oc], rbase + ccl, mask=m)

    def finish_unit(slot_rid, par):
        for t, (k, col0, npx, r0, chunks, nlo, nhi) in enumerate(tconst):
            lo = _STARTS[t]
            hi = _STARTS[t + 1] if t + 1 < len(_STARTS) else _NUNITS
            u = unit_u(slot_rid, t)
            srow = r0 + u
            valid = jnp.logical_and(srow >= 0, srow < _H)
            on = jnp.logical_and(slot_rid >= lo, slot_rid < hi)
            base = _SIZES[k] * u + col0
            buf = bufs[par]

            @pl.when(jnp.logical_and(on, valid))
            def _():
                def zlo(p, carry):
                    pltpu.sync_copy(zin.at[pl.ds(0, 1)],
                                    buf.at[pl.ds(p, 1)])
                    return carry

                def zhi(p, carry):
                    pltpu.sync_copy(zin.at[pl.ds(0, 1)],
                                    buf.at[pl.ds(npx - 1 - p, 1)])
                    return carry

                lax.fori_loop(0, nlo, zlo, jnp.int32(0))
                lax.fori_loop(0, nhi, zhi, jnp.int32(0))
                pltpu.async_copy(buf.at[pl.ds(0, npx)],
                                 outs[k].at[pl.ds(base, npx)], sem_o).wait()

            @pl.when(jnp.logical_and(on, jnp.logical_not(valid)))
            def _():
                off = 0
                while off < npx:
                    n = min(_LANES, npx - off)
                    pltpu.sync_copy(zer.at[pl.ds(0, n)],
                                    outs[k].at[pl.ds(base + off, n)])
                    off += n

    nslots = (_NUNITS + nworkers - 1) // nworkers
    rids = [jnp.minimum(wid + s * nworkers, _NUNITS - 1)
            for s in range(nslots)]

    build_idx(rids[0], 0)
    g = pltpu.async_copy(ffr.at[idxs[0]], bufs[0], gsems[0])
    for s in range(nslots):
        par = s % 2
        nxt = (s + 1) % 2
        g.wait()
        if s + 1 < nslots:
            build_idx(rids[s + 1], nxt)
            g = pltpu.async_copy(ffr.at[idxs[nxt]], bufs[nxt], gsems[nxt])
        finish_unit(rids[s], par)

    @pl.when(wid == nworkers - 1)
    def _():
        s25 = _HC * r + c
        pltpu.sync_copy(cor.at[pl.ds(s25, 1)], bufa.at[pl.ds(0, 1)])
        pltpu.sync_copy(bufa.at[pl.ds(0, 1)], q3)


def _tc_body(anch_ref, ffr4, z_r, z_c, q04, sem):
    n = _SIZES[0]
    r = anch_ref[0]
    c = anch_ref[1]
    r0 = 4 * r - _PADS[0]
    c0 = 4 * c - _PADS[0]
    rlo = jnp.clip(-r0, 0, n)
    rhi = jnp.clip(r0 + n - _H, 0, n)
    clo = jnp.clip(-c0, 0, n)
    chi = jnp.clip(c0 + n - _H, 0, n)
    nrows = n - rlo - rhi
    ncols = n - clo - chi
    sr0 = jnp.maximum(r0, 0)
    sc0 = jnp.maximum(c0, 0)

    sizes = (n, n - 4, n - 8, n - 12, n - 16)
    for nrv in sizes:
        for ncv in sizes:

            @pl.when(jnp.logical_and(nrows == nrv, ncols == ncv))
            def _(nrv=nrv, ncv=ncv):
                cp = pltpu.make_async_copy(
                    ffr4.at[pl.ds(sr0, nrv), pl.ds(sc0, ncv)],
                    q04.at[pl.ds(rlo, nrv), pl.ds(clo, ncv)], sem)
                cp.start()
                cp.wait()

    for v in (4, 8, 12, 16):

        @pl.when(rlo == v)
        def _(v=v):
            cp = pltpu.make_async_copy(
                z_r.at[pl.ds(0, v)], q04.at[pl.ds(0, v)], sem)
            cp.start()
            cp.wait()

        @pl.when(rhi == v)
        def _(v=v):
            cp = pltpu.make_async_copy(
                z_r.at[pl.ds(0, v)], q04.at[pl.ds(n - v, v)], sem)
            cp.start()
            cp.wait()

        @pl.when(clo == v)
        def _(v=v):
            cp = pltpu.make_async_copy(
                z_c.at[pl.ds(0, n), pl.ds(0, v)],
                q04.at[pl.ds(0, n), pl.ds(0, v)], sem)
            cp.start()
            cp.wait()

        @pl.when(chi == v)
        def _(v=v):
            cp = pltpu.make_async_copy(
                z_c.at[pl.ds(0, n), pl.ds(0, v)],
                q04.at[pl.ds(0, n), pl.ds(n - v, v)], sem)
            cp.start()
            cp.wait()


def kernel(full_feature, corr_feature, anchor):
    B, C, H, W = full_feature.shape
    ffr = _to_rows(full_feature)
    cor = _to_rows(corr_feature)
    a32 = anchor.astype(jnp.int32)
    anch = jnp.concatenate([
        jnp.broadcast_to(a32[0], (_LANES,)),
        jnp.broadcast_to(a32[1], (_LANES,)),
    ])
    f32 = jnp.float32
    zin = jnp.asarray(np.zeros((_LANES, 8, 128), np.float32))

    mesh = plsc.VectorSubcoreMesh(core_axis_name="c", subcore_axis_name="s")
    nworkers = mesh.num_cores * mesh.num_subcores

    run = pl.kernel(
        functools.partial(_body, nsub=mesh.num_subcores, nworkers=nworkers),
        out_type=(
            jax.ShapeDtypeStruct((31 * 31, 8, 128), f32),
            jax.ShapeDtypeStruct((15 * 15, 8, 128), f32),
            jax.ShapeDtypeStruct((1, 8, 128), f32),
        ),
        mesh=mesh,
        compiler_params=pltpu.CompilerParams(use_tc_tiling_on_sc=True,
                                             needs_layout_passes=False),
        scratch_types=[
            pltpu.VMEM((2 * _LANES,), jnp.int32),
            pltpu.VMEM((_U, 8, 128), f32),
            pltpu.VMEM((_U, 8, 128), f32),
            pltpu.VMEM((_LANES, 8, 128), f32),
            pltpu.VMEM((_U,), jnp.int32),
            pltpu.VMEM((_U,), jnp.int32),
            pltpu.SemaphoreType.DMA,
            pltpu.SemaphoreType.DMA,
            pltpu.SemaphoreType.DMA,
            pltpu.SemaphoreType.DMA,
        ],
    )
    q1, q2, q3 = run(ffr, cor, anch, zin)

    ffr4 = ffr.reshape(_H, _H, 8, 128)
    z_r = jnp.asarray(np.zeros((_PADS[0], _SIZES[0], 8, 128), np.float32))
    z_c = jnp.asarray(np.zeros((_SIZES[0], _PADS[0], 8, 128), np.float32))
    grid_spec = pltpu.PrefetchScalarGridSpec(
        num_scalar_prefetch=1,
        grid=(1,),
        in_specs=[pl.BlockSpec(memory_space=pl.ANY)] * 3,
        out_specs=pl.BlockSpec(memory_space=pl.ANY),
        scratch_shapes=[pltpu.SemaphoreType.DMA],
    )
    q0 = pl.pallas_call(
        _tc_body,
        grid_spec=grid_spec,
        out_shape=jax.ShapeDtypeStruct((_SIZES[0], _SIZES[0], 8, 128), f32),
    )(a32, ffr4, z_r, z_c)
    q0 = q0.reshape(_SIZES[0] * _SIZES[0], 8, 128)

    return (_from_rows(q0, 61, 61), _from_rows(q1, 31, 31),
            _from_rows(q2, 15, 15), _from_rows(q3, 1, 1))


# speedup vs baseline: 15.4861x; 10.3879x over previous
import functools

import jax
import jax.numpy as jnp
import numpy as np
from jax import lax
from jax.experimental import pallas as pl
from jax.experimental.pallas import tpu as pltpu
from jax.experimental.pallas import tpu_sc as plsc

_H = 125
_HC = 25
_LANES = 16
_U = 61
_US = 31
_SIZES = (61, 31, 15)
_SCALES = (4, 2, 1)
_PADS = (16, 8, 4)
_TYPES = ((0, 0, 61), (1, 0, 31), (2, 0, 15))
_STARTS = (0, 61, 92)
_NUNITS = 107


def _to_rows(x):
    B, C, H, W = x.shape
    y = x.reshape(B, 2, 128, H, W).transpose(3, 4, 1, 0, 2)
    return y.reshape(H * W, 8, 128)


def _from_rows(y, H, W):
    z = y.reshape(H, W, 2, 4, 128).transpose(3, 2, 4, 0, 1)
    return z.reshape(4, 256, H, W)


def _body(ffr, cor, anch, zin, q0, q1, q2, q3,
          av, bufa, bufb, zer, idxa, idxb,
          sem_ga, sem_gb, sem_o, sem_m, *, nsub, nworkers):
    outs = {0: q0, 1: q1, 2: q2}
    bufs = (bufa, bufb)
    idxs = (idxa, idxb)
    gsems = (sem_ga, sem_gb)

    wid = lax.axis_index("c") * nsub + lax.axis_index("s")

    pltpu.sync_copy(zin, zer)

    pltpu.sync_copy(anch, av)
    lane = lax.iota(jnp.int32, _LANES)
    r = jnp.max(plsc.load_gather(av, [lane]))
    c = jnp.max(plsc.load_gather(av, [_LANES + lane]))

    zero16 = jnp.broadcast_to(jnp.int32(0), (_LANES,))
    for ib, ulen in ((idxa, _U), (idxb, _US)):
        for j in range((ulen + _LANES - 1) // _LANES):
            off = _LANES * j + lane
            if _LANES * (j + 1) <= ulen:
                plsc.store_scatter(ib, [off], zero16)
            else:
                plsc.store_scatter(ib, [off], zero16, mask=off < ulen)

    tconst = []
    for (k, col0, npx) in _TYPES:
        size, scale, pad = _SIZES[k], _SCALES[k], _PADS[k]
        r0 = scale * r - pad
        c0 = scale * c - pad
        chunks = []
        for j in range((npx + _LANES - 1) // _LANES):
            oc = _LANES * j + lane
            chunks.append((oc, jnp.clip(c0 + col0 + oc, 0, _H - 1),
                           oc < npx))
        nlo = jnp.clip(-c0 - col0, 0, npx)
        nhi = jnp.clip(c0 + col0 + npx - _H, 0, npx)
        tconst.append((k, col0, npx, r0, chunks, nlo, nhi))

    def unit_u(rid, t):
        return rid - _STARTS[t]

    def build_idx(slot_rid, par):
        for t, (k, col0, npx, r0, chunks, nlo, nhi) in enumerate(tconst):
            if (par == 0) != (t == 0):
                continue
            lo = _STARTS[t]
            hi = _STARTS[t + 1] if t + 1 < len(_STARTS) else _NUNITS
            u = unit_u(slot_rid, t)
            srow = r0 + u
            valid = jnp.logical_and(srow >= 0, srow < _H)
            on = jnp.logical_and(
                jnp.logical_and(slot_rid >= lo, slot_rid < hi), valid)

            @pl.when(on)
            def _():
                rbase = srow * _H
                for (oc, ccl, m) in chunks:
                    plsc.store_scatter(idxs[par], [oc], rbase + ccl, mask=m)

    def finish_unit(slot_rid, par):
        for t, (k, col0, npx, r0, chunks, nlo, nhi) in enumerate(tconst):
            if (par == 0) != (t == 0):
                continue
            lo = _STARTS[t]
            hi = _STARTS[t + 1] if t + 1 < len(_STARTS) else _NUNITS
            u = unit_u(slot_rid, t)
            srow = r0 + u
            valid = jnp.logical_and(srow >= 0, srow < _H)
            on = jnp.logical_and(slot_rid >= lo, slot_rid < hi)
            base = _SIZES[k] * u + col0
            buf = bufs[par]

            @pl.when(jnp.logical_and(on, valid))
            def _():
                def zlo(p, carry):
                    pltpu.sync_copy(zin.at[pl.ds(0, 1)],
                                    buf.at[pl.ds(p, 1)])
                    return carry

                def zhi(p, carry):
                    pltpu.sync_copy(zin.at[pl.ds(0, 1)],
                                    buf.at[pl.ds(npx - 1 - p, 1)])
                    return carry

                lax.fori_loop(0, nlo, zlo, jnp.int32(0))
                lax.fori_loop(0, nhi, zhi, jnp.int32(0))
                pltpu.async_copy(buf.at[pl.ds(0, npx)],
                                 outs[k].at[pl.ds(base, npx)], sem_o).wait()

            @pl.when(jnp.logical_and(on, jnp.logical_not(valid)))
            def _():
                off = 0
                while off < npx:
                    n = min(_LANES, npx - off)
                    pltpu.sync_copy(zer.at[pl.ds(0, n)],
                                    outs[k].at[pl.ds(base + off, n)])
                    off += n

    nslots = 4
    rids = [jnp.minimum(wid, _STARTS[1] - 1),
            _STARTS[1] + wid,
            jnp.minimum(nworkers + wid, _STARTS[1] - 1),
            jnp.minimum(_STARTS[1] + nworkers + wid, _NUNITS - 1)]

    build_idx(rids[0], 0)
    g = pltpu.async_copy(ffr.at[idxs[0]], bufs[0], gsems[0])
    for s in range(nslots):
        par = s % 2
        nxt = (s + 1) % 2
        g.wait()
        if s + 1 < nslots:
            build_idx(rids[s + 1], nxt)
            g = pltpu.async_copy(ffr.at[idxs[nxt]], bufs[nxt], gsems[nxt])
        finish_unit(rids[s], par)

    @pl.when(wid == nworkers - 1)
    def _():
        s25 = _HC * r + c
        pltpu.sync_copy(cor.at[pl.ds(s25, 1)], bufa.at[pl.ds(0, 1)])
        pltpu.sync_copy(bufa.at[pl.ds(0, 1)], q3)


def kernel(full_feature, corr_feature, anchor):
    B, C, H, W = full_feature.shape
    ffr = _to_rows(full_feature)
    cor = _to_rows(corr_feature)
    a32 = anchor.astype(jnp.int32)
    anch = jnp.concatenate([
        jnp.broadcast_to(a32[0], (_LANES,)),
        jnp.broadcast_to(a32[1], (_LANES,)),
    ])
    f32 = jnp.float32
    zin = jnp.asarray(np.zeros((_LANES, 8, 128), np.float32))

    mesh = plsc.VectorSubcoreMesh(core_axis_name="c", subcore_axis_name="s")
    nworkers = mesh.num_cores * mesh.num_subcores

    run = pl.kernel(
        functools.partial(_body, nsub=mesh.num_subcores, nworkers=nworkers),
        out_type=(
            jax.ShapeDtypeStruct((61 * 61, 8, 128), f32),
            jax.ShapeDtypeStruct((31 * 31, 8, 128), f32),
            jax.ShapeDtypeStruct((15 * 15, 8, 128), f32),
            jax.ShapeDtypeStruct((1, 8, 128), f32),
        ),
        mesh=mesh,
        compiler_params=pltpu.CompilerParams(use_tc_tiling_on_sc=True,
                                             needs_layout_passes=False),
        scratch_types=[
            pltpu.VMEM((2 * _LANES,), jnp.int32),
            pltpu.VMEM((_U, 8, 128), f32),
            pltpu.VMEM((_US, 8, 128), f32),
            pltpu.VMEM((_LANES, 8, 128), f32),
            pltpu.VMEM((_U,), jnp.int32),
            pltpu.VMEM((_US,), jnp.int32),
            pltpu.SemaphoreType.DMA,
            pltpu.SemaphoreType.DMA,
            pltpu.SemaphoreType.DMA,
            pltpu.SemaphoreType.DMA,
        ],
    )
    q0, q1, q2, q3 = run(ffr, cor, anch, zin)

    return (_from_rows(q0, 61, 61), _from_rows(q1, 31, 31),
            _from_rows(q2, 15, 15), _from_rows(q3, 1, 1))
